# Initial kernel scaffold; baseline (speedup 1.0000x reference)
import jax
import jax.numpy as jnp
from jax.experimental import pallas as pl


def kernel(c, conv_3_3, conv_4_3, conv_5_3):
    raise NotImplementedError("write your pallas kernel here")


# speedup vs baseline: 54713.0928x; 54713.0928x over previous
import functools

import jax
import jax.numpy as jnp
from jax import lax
from jax.experimental import pallas as pl
from jax.experimental.pallas import tpu as pltpu
from jax.experimental.pallas import tpu_sc as plsc

NC = 2
NS = 16
L = 16
NW = NC * NS


def _make_sc_kernel(B, V, scales):
  ROWS = B * V
  assert ROWS % NW == 0
  rpw = ROWS // NW
  assert rpw % L == 0
  n_chunks = rpw // L
  C_TOT = sum(c for (_, _, c, _) in scales)

  mesh = plsc.VectorSubcoreMesh(
      core_axis_name="c", subcore_axis_name="s",
      num_cores=NC, num_subcores=NS)

  scratch = [
      pltpu.VMEM((rpw,), jnp.float32),
      pltpu.VMEM((rpw,), jnp.float32),
      pltpu.VMEM((4 * L,), jnp.int32),
      [pltpu.VMEM((4 * L, c), jnp.float32) for (_, _, c, _) in scales],
      pltpu.VMEM((L, C_TOT), jnp.float32),
      pltpu.SemaphoreType.DMA,
  ]

  @functools.partial(
      pl.kernel,
      mesh=mesh,
      out_type=jax.ShapeDtypeStruct((ROWS, C_TOT), jnp.float32),
      scratch_types=scratch,
  )
  def k(cx_hbm, cy_hbm, t0_hbm, t1_hbm, t2_hbm, out_hbm,
        cx_v, cy_v, idx_v, gbufs, obuf, sem):
    tables = (t0_hbm, t1_hbm, t2_hbm)
    wid = lax.axis_index("s") * NC + lax.axis_index("c")
    base = wid * rpw
    batch = base // V

    pltpu.sync_copy(cx_hbm.at[pl.ds(base, rpw)], cx_v)
    pltpu.sync_copy(cy_hbm.at[pl.ds(base, rpw)], cy_v)

    for ch in range(n_chunks):
      co = ch * L
      coff = 0
      for s, (H, W, C, inv) in enumerate(scales):
        x = cx_v[pl.ds(co, L)] * inv
        y = cy_v[pl.ds(co, L)] * inv
        x1i = x.astype(jnp.int32)
        y1i = y.astype(jnp.int32)
        x1f = x1i.astype(jnp.float32)
        y1f = y1i.astype(jnp.float32)
        one = jnp.full((L,), 1, jnp.int32)
        zero = jnp.full((L,), 0, jnp.int32)
        x2i = x1i + jnp.where(x > x1f, one, zero)
        y2i = y1i + jnp.where(y > y1f, one, zero)
        wx2 = x - x1f
        wx1 = x2i.astype(jnp.float32) - x
        wy2 = y - y1f
        wy1 = y2i.astype(jnp.float32) - y
        boff = batch * (H * W)
        r1 = y1i * W + boff
        r2 = y2i * W + boff
        idx_v[pl.ds(0 * L, L)] = r1 + x1i
        idx_v[pl.ds(1 * L, L)] = r2 + x1i
        idx_v[pl.ds(2 * L, L)] = r1 + x2i
        idx_v[pl.ds(3 * L, L)] = r2 + x2i
        w11 = wx1 * wy1
        w12 = wx1 * wy2
        w21 = wx2 * wy1
        w22 = wx2 * wy2

        gbuf = gbufs[s]
        pltpu.async_copy(tables[s].at[idx_v], gbuf, sem).wait()

        dn = lax.GatherDimensionNumbers(
            offset_dims=(), collapsed_slice_dims=(0,), start_index_map=(0,))

        def _splat(vec, sp, dn=dn):
          return lax.gather(
              vec, sp[:, None], dn, (1,),
              mode=lax.GatherScatterMode.PROMISE_IN_BOUNDS)

        def row_body(r, _, C=C, coff=coff, gbuf=gbuf,
                     w11=w11, w12=w12, w21=w21, w22=w22):
          sp = jnp.full((L,), 0, jnp.int32) + r
          w0 = _splat(w11, sp)
          w1 = _splat(w12, sp)
          w2 = _splat(w21, sp)
          w3 = _splat(w22, sp)

          def ch_body(j, _, r=r):
            acc = w0 * gbuf[0 * L + r, pl.ds(j * L, L)]
            acc += w1 * gbuf[1 * L + r, pl.ds(j * L, L)]
            acc += w2 * gbuf[2 * L + r, pl.ds(j * L, L)]
            acc += w3 * gbuf[3 * L + r, pl.ds(j * L, L)]
            obuf[r, pl.ds(coff + j * L, L)] = acc
            return _

          return lax.fori_loop(0, C // L, ch_body, _)

        lax.fori_loop(0, L, row_body, 0)
        coff += C

      pltpu.sync_copy(obuf, out_hbm.at[pl.ds(base + co, L)])

  return k


def kernel(c, conv_3_3, conv_4_3, conv_5_3):
  B, V, _ = c.shape
  maps = (conv_3_3, conv_4_3, conv_5_3)
  scales = []
  inv = 1.0 / 8.0
  for fm in maps:
    _, C, H, W = fm.shape
    scales.append((H, W, C, inv))
    inv *= 0.5

  cx = c[:, :, 0].reshape(-1)
  cy = c[:, :, 1].reshape(-1)
  tables = [fm.transpose(0, 2, 3, 1).reshape(-1, fm.shape[1]) for fm in maps]

  k = _make_sc_kernel(B, V, tuple(scales))
  out = k(cx, cy, *tables)
  return out.reshape(B, V, out.shape[-1])


# speedup vs baseline: 88907.7260x; 1.6250x over previous
import functools

import jax
import jax.numpy as jnp
from jax import lax
from jax.experimental import pallas as pl
from jax.experimental.pallas import tpu as pltpu
from jax.experimental.pallas import tpu_sc as plsc

NC = 2
NS = 16
L = 16
NW = NC * NS


def _make_sc_kernel(B, V, scales):
  ROWS = B * V
  assert ROWS % NW == 0
  rpw = ROWS // NW
  assert rpw % L == 0
  n_chunks = rpw // L
  C_TOT = sum(c for (_, _, c, _) in scales)

  mesh = plsc.VectorSubcoreMesh(
      core_axis_name="c", subcore_axis_name="s",
      num_cores=NC, num_subcores=NS)

  scratch = [
      pltpu.VMEM((rpw,), jnp.float32),
      pltpu.VMEM((rpw,), jnp.float32),
      [pltpu.VMEM((4 * L,), jnp.int32) for _ in scales],
      [pltpu.VMEM((4 * L, c), jnp.float32) for (_, _, c, _) in scales],
      pltpu.VMEM((L, C_TOT), jnp.float32),
      [pltpu.SemaphoreType.DMA for _ in scales],
  ]

  @functools.partial(
      pl.kernel,
      mesh=mesh,
      out_type=jax.ShapeDtypeStruct((ROWS, C_TOT), jnp.float32),
      scratch_types=scratch,
  )
  def k(cx_hbm, cy_hbm, t0_hbm, t1_hbm, t2_hbm, out_hbm,
        cx_v, cy_v, idx_vs, gbufs, obuf, sems):
    tables = (t0_hbm, t1_hbm, t2_hbm)
    wid = lax.axis_index("s") * NC + lax.axis_index("c")
    base = wid * rpw
    batch = base // V

    pltpu.sync_copy(cx_hbm.at[pl.ds(base, rpw)], cx_v)
    pltpu.sync_copy(cy_hbm.at[pl.ds(base, rpw)], cy_v)

    def corner_geom(ch, s):
      H, W, C, inv = scales[s]
      x = cx_v[pl.ds(ch * L, L)] * inv
      y = cy_v[pl.ds(ch * L, L)] * inv
      x1i = x.astype(jnp.int32)
      y1i = y.astype(jnp.int32)
      x1f = x1i.astype(jnp.float32)
      y1f = y1i.astype(jnp.float32)
      one = jnp.full((L,), 1, jnp.int32)
      zero = jnp.full((L,), 0, jnp.int32)
      x2i = x1i + jnp.where(x > x1f, one, zero)
      y2i = y1i + jnp.where(y > y1f, one, zero)
      return x, y, x1i, y1i, x1f, y1f, x2i, y2i

    def fire(ch, s):
      H, W, C, inv = scales[s]
      _, _, x1i, y1i, _, _, x2i, y2i = corner_geom(ch, s)
      idx_v = idx_vs[s]
      r1 = y1i * W + batch * (H * W)
      r2 = y2i * W + batch * (H * W)
      idx_v[pl.ds(0 * L, L)] = r1 + x1i
      idx_v[pl.ds(1 * L, L)] = r2 + x1i
      idx_v[pl.ds(2 * L, L)] = r1 + x2i
      idx_v[pl.ds(3 * L, L)] = r2 + x2i
      return pltpu.async_copy(tables[s].at[idx_v], gbufs[s], sems[s])

    dn = lax.GatherDimensionNumbers(
        offset_dims=(), collapsed_slice_dims=(0,), start_index_map=(0,))

    def splat(vec, sp):
      return lax.gather(vec, sp[:, None], dn, (1,),
                        mode=lax.GatherScatterMode.PROMISE_IN_BOUNDS)

    def compute(ch, s, coff):
      H, W, C, inv = scales[s]
      x, y, _, y1i, x1f, y1f, x2i, y2i = corner_geom(ch, s)
      wx2 = x - x1f
      wx1 = x2i.astype(jnp.float32) - x
      wy2 = y - y1f
      wy1 = y2i.astype(jnp.float32) - y
      w11 = wx1 * wy1
      w12 = wx1 * wy2
      w21 = wx2 * wy1
      w22 = wx2 * wy2
      gbuf = gbufs[s]

      @plsc.parallel_loop(0, L)
      def row_body(r):
        sp = jnp.full((L,), 0, jnp.int32) + r
        w0 = splat(w11, sp)
        w1 = splat(w12, sp)
        w2 = splat(w21, sp)
        w3 = splat(w22, sp)

        @plsc.parallel_loop(0, C // L, unroll=4)
        def ch_body(j):
          acc = w0 * gbuf[0 * L + r, pl.ds(j * L, L)]
          acc += w1 * gbuf[1 * L + r, pl.ds(j * L, L)]
          acc += w2 * gbuf[2 * L + r, pl.ds(j * L, L)]
          acc += w3 * gbuf[3 * L + r, pl.ds(j * L, L)]
          obuf[r, pl.ds(coff + j * L, L)] = acc

    handles = [fire(0, s) for s in range(len(scales))]
    for ch in range(n_chunks):
      coff = 0
      for s in range(len(scales)):
        handles[s].wait()
        compute(ch, s, coff)
        if ch + 1 < n_chunks:
          handles[s] = fire(ch + 1, s)
        coff += scales[s][2]
      pltpu.sync_copy(obuf, out_hbm.at[pl.ds(base + ch * L, L)])

  return k


def kernel(c, conv_3_3, conv_4_3, conv_5_3):
  B, V, _ = c.shape
  maps = (conv_3_3, conv_4_3, conv_5_3)
  scales = []
  inv = 1.0 / 8.0
  for fm in maps:
    _, C, H, W = fm.shape
    scales.append((H, W, C, inv))
    inv *= 0.5

  cx = c[:, :, 0].reshape(-1)
  cy = c[:, :, 1].reshape(-1)
  tables = [fm.transpose(0, 2, 3, 1).reshape(-1, fm.shape[1]) for fm in maps]

  k = _make_sc_kernel(B, V, tuple(scales))
  out = k(cx, cy, *tables)
  return out.reshape(B, V, out.shape[-1])
